# baseline (speedup 1.0000x reference)
import jax
import jax.numpy as jnp
from jax.experimental import pallas as pl

BLOCK_M = 4096
LANES = 128


def _vq_kernel(x_ref, embed_ref, out_ref):
    bb, t, d = x_ref.shape
    m = bb * t
    x = x_ref[...].reshape(m, d)
    embed = embed_ref[...]
    k = embed.shape[0]
    nh = jnp.sum(embed * embed, axis=1, keepdims=True) * -0.5
    e_aug = jnp.concatenate([embed, nh], axis=1)
    x_aug = jnp.concatenate([x, jnp.ones((m, 1), jnp.float32)], axis=1)
    score = jax.lax.dot_general(
        x_aug, e_aug,
        dimension_numbers=(((1,), (1,)), ((), ())),
        preferred_element_type=jnp.float32,
    )
    lane_f = jax.lax.broadcasted_iota(jnp.int32, (m, LANES), 1).astype(jnp.float32)
    rmax = score[:, 0:LANES]
    ridx = lane_f
    for c in range(1, k // LANES):
        lo = c * LANES
        sc = score[:, lo:lo + LANES]
        upd = sc > rmax
        ridx = jnp.where(upd, lane_f + float(lo), ridx)
        rmax = jnp.maximum(rmax, sc)
    mbest = jnp.max(rmax, axis=1, keepdims=True)
    cand = jnp.where(rmax == mbest, ridx, float(k))
    idx = jnp.min(cand, axis=1, keepdims=True)
    k_iota = jax.lax.broadcasted_iota(jnp.int32, (m, k), 1).astype(jnp.float32)
    onehot = (k_iota == idx).astype(jnp.float32)
    quant = jax.lax.dot_general(
        onehot, embed,
        dimension_numbers=(((1,), (0,)), ((), ())),
        preferred_element_type=jnp.float32,
    )
    out_ref[...] = quant.reshape(bb, t, d)


def kernel(x, embed):
    b, t, d = x.shape
    bb = BLOCK_M // t
    grid = (b // bb,)
    quant = pl.pallas_call(
        _vq_kernel,
        grid=grid,
        in_specs=[
            pl.BlockSpec((bb, t, d), lambda i: (i, 0, 0)),
            pl.BlockSpec(embed.shape, lambda i: (0, 0)),
        ],
        out_specs=pl.BlockSpec((bb, t, d), lambda i: (i, 0, 0)),
        out_shape=jax.ShapeDtypeStruct((b, t, d), jnp.float32),
    )(x, embed)
    return (quant, 0)


# speedup vs baseline: 1.1646x; 1.1646x over previous
import jax
import jax.numpy as jnp
from jax.experimental import pallas as pl

BLOCK_M = 8192
LANES = 128


def _vq_kernel(x_ref, embed_t2_ref, norms_ref, embed_ref, out_ref):
    bb, t, d = x_ref.shape
    m = bb * t
    x = x_ref[...].reshape(m, d)
    embed_t2 = embed_t2_ref[...]
    embed = embed_ref[...]
    dots = jax.lax.dot_general(
        x, embed_t2,
        dimension_numbers=(((1,), (0,)), ((), ())),
        preferred_element_type=jnp.float32,
    )
    k = dots.shape[1]
    norms = norms_ref[...]
    lane_f = jax.lax.broadcasted_iota(jnp.int32, (m, LANES), 1).astype(jnp.float32)
    rmin = dots[:, 0:LANES] + norms[:, 0:LANES]
    ridx = lane_f
    for c in range(1, k // LANES):
        lo = c * LANES
        dc = dots[:, lo:lo + LANES] + norms[:, lo:lo + LANES]
        upd = dc < rmin
        ridx = jnp.where(upd, lane_f + float(lo), ridx)
        rmin = jnp.minimum(rmin, dc)
    mdist = jnp.min(rmin, axis=1, keepdims=True)
    cand = jnp.where(rmin == mdist, ridx, float(k))
    idx = jnp.min(cand, axis=1, keepdims=True)
    k_iota = jax.lax.broadcasted_iota(jnp.int32, (m, k), 1).astype(jnp.float32)
    onehot = (k_iota == idx).astype(jnp.float32)
    quant = jax.lax.dot_general(
        onehot, embed,
        dimension_numbers=(((1,), (0,)), ((), ())),
        preferred_element_type=jnp.float32,
    )
    out_ref[...] = quant.reshape(bb, t, d)


def kernel(x, embed):
    b, t, d = x.shape
    bb = BLOCK_M // t
    embed_t2 = -2.0 * embed.T
    norms = jnp.sum(embed * embed, axis=1)[None, :]
    grid = (b // bb,)
    quant = pl.pallas_call(
        _vq_kernel,
        grid=grid,
        in_specs=[
            pl.BlockSpec((bb, t, d), lambda i: (i, 0, 0)),
            pl.BlockSpec(embed_t2.shape, lambda i: (0, 0)),
            pl.BlockSpec(norms.shape, lambda i: (0, 0)),
            pl.BlockSpec(embed.shape, lambda i: (0, 0)),
        ],
        out_specs=pl.BlockSpec((bb, t, d), lambda i: (i, 0, 0)),
        out_shape=jax.ShapeDtypeStruct((b, t, d), jnp.float32),
    )(x, embed_t2, norms, embed)
    return (quant, 0)
